# baseline (speedup 1.0000x reference)
import functools

import jax
import jax.numpy as jnp
from jax import lax
from jax.experimental import pallas as pl
from jax.experimental.pallas import tpu as pltpu
from jax.experimental.pallas import tpu_sc as plsc

B = 16384
D = 64
H = 128
NV = 1000000
NP = 100000

NC = 2
NS = 16
NW = NC * NS

LANES = 128
TV = (NV + LANES - 1) // LANES
TP = (NP + LANES - 1) // LANES
TV_Q, TV_R = divmod(TV, NW)
TP_Q, TP_R = divmod(TP, NW)
NT_V = TV_Q + 1
NT_P = TP_Q + 1

KP = 64
KV = 26
BIN_P = 0
BIN_V = NT_P * KP
BINS = BIN_V + NT_V * KV
TRASH = BINS

HITCAP_P = 768
HITCAP_V = 1408
HITCAP = HITCAP_P + HITCAP_V

CNT_P = 0
CNT_V = NT_P + 8
NCNT = CNT_V + NT_V + 8

OUT_ROWS = 3 * B + LANES
RING = 3
GRP = 4
SCH = 2048
SROWS = 56


def _splat(x):
    return jnp.full((16,), x, jnp.int32)


def _sc_sweep_body(pidx_hbm, widx_hbm, lidx_hbm, pt_hbm, vt_hbm, out_hbm,
                   idxv, panels, stage, posb, hit_t, hit_lp, bins, cnt,
                   sem_idx, sem_ring, sem_flush):
    wid = lax.axis_index("s") * NC + lax.axis_index("c")
    t0v = wid * TV_Q + jnp.minimum(wid, TV_R)
    ntv = TV_Q + (wid < TV_R).astype(jnp.int32)
    t0p = wid * TP_Q + jnp.minimum(wid, TP_R)
    ntp = TP_Q + (wid < TP_R).astype(jnp.int32)

    for g in range(RING):
        pltpu.async_copy(
            pt_hbm.at[:, pl.ds(jnp.minimum(t0p + g * GRP, TP - GRP) * LANES,
                               GRP * LANES)],
            panels.at[g], sem_ring)

    def zcnt(i, _):
        cnt[i] = 0
        return ()
    lax.fori_loop(0, NCNT, zcnt, ())

    def scan(src_hbm, t0, nt, pos_base, off0):
        pltpu.async_copy(src_hbm.at[pl.ds(0, SCH)], idxv.at[pl.ds(0, SCH)],
                         sem_idx)

        def chunk(cix, off):
            pltpu.make_async_copy(src_hbm.at[pl.ds(0, SCH)],
                                  idxv.at[pl.ds(0, SCH)], sem_idx).wait()

            @pl.when(cix + 1 < B // SCH)
            def _():
                pltpu.async_copy(
                    src_hbm.at[pl.ds((cix + 1) * SCH, SCH)],
                    idxv.at[pl.ds(lax.rem(cix + 1, 2) * SCH, SCH)], sem_idx)
            cbase = lax.rem(cix, 2) * SCH

            def body(k, off):
                iv0 = idxv[pl.ds(cbase + k * 32, 16)]
                iv1 = idxv[pl.ds(cbase + k * 32 + 16, 16)]
                t_0 = lax.shift_right_logical(iv0, 7)
                t_1 = lax.shift_right_logical(iv1, 7)
                m0 = (t_0 >= t0) & (t_0 < t0 + nt)
                m1 = (t_1 >= t0) & (t_1 < t0 + nt)
                c0 = plsc.cumsum(m0.astype(jnp.int32))
                c1 = plsc.cumsum(m1.astype(jnp.int32))
                base = pos_base + cix * SCH + k * 32 + lax.iota(jnp.int32, 16)
                pos0 = jnp.where(m0, jnp.minimum(off + c0 - 1, HITCAP), HITCAP)
                plsc.store_scatter(hit_t, [pos0], t_0)
                plsc.store_scatter(hit_lp, [pos0],
                                   lax.bitwise_and(iv0, LANES - 1) + (base << 7))
                off1 = off + c0[15]
                pos1 = jnp.where(m1, jnp.minimum(off1 + c1 - 1, HITCAP), HITCAP)
                plsc.store_scatter(hit_t, [pos1], t_1)
                plsc.store_scatter(hit_lp, [pos1],
                                   lax.bitwise_and(iv1, LANES - 1)
                                   + ((base + 16) << 7))
                return off1 + c1[15]
            return lax.fori_loop(0, SCH // 32, body, off)
        return lax.fori_loop(0, B // SCH, chunk, off0)

    nh_p = scan(pidx_hbm, t0p, ntp, 0, 0)
    nh_w = scan(widx_hbm, t0v, ntv, B, HITCAP_P)
    nh_v = scan(lidx_hbm, t0v, ntv, 2 * B, nh_w)

    def binhits(lo, hi, t0, cbase, bbase, K):
        def body(h, _):
            t = hit_t[pl.ds(h, 16)][0]
            lp = hit_lp[pl.ds(h, 16)][0]
            r = t - t0
            slot = cnt[cbase + r]
            cnt[cbase + r] = slot + 1
            addr = jnp.where(slot < K, bbase + r * K + slot, TRASH)
            plsc.store_scatter(bins, [_splat(addr)], _splat(lp))
            return ()
        lax.fori_loop(lo, hi, body, ())

    binhits(0, nh_p, t0p, CNT_P, BIN_P, KP)
    binhits(HITCAP_P, nh_v, t0v, CNT_V, BIN_V, KV)

    def reset_posb(c):
        for j in range(SROWS // 16):
            posb[c, pl.ds(16 * j, 16)] = _splat(3 * B + 16 * j) + lax.iota(jnp.int32, 16)
    reset_posb(0)
    reset_posb(1)

    def flush_wait():
        pltpu.make_async_copy(
            pt_hbm.at[pl.ds(0, SROWS), pl.ds(0, LANES)], stage.at[0],
            sem_flush).wait()

    def sweep(table, t0, nt, tmax, cbase, bbase, K, st, primed=False):
        ngrp = (nt + GRP - 1) // GRP

        def gbase(g):
            return jnp.minimum(t0 + g * GRP, tmax - GRP)

        def fire(g):
            @pl.when(g < ngrp)
            def _():
                pltpu.async_copy(
                    table.at[:, pl.ds(gbase(g) * LANES, GRP * LANES)],
                    panels.at[lax.rem(g, RING)], sem_ring)
        if not primed:
            fire(0)
            fire(1)
            fire(2)

        def grp_body(g, st):
            pltpu.make_async_copy(
                table.at[:, pl.ds(0, GRP * LANES)], panels.at[0],
                sem_ring).wait()
            gmod = lax.rem(g, RING)
            gb = gbase(g)

            def tile_body(tt, st):
                t_rel = g * GRP + tt
                t_abs = t0 + t_rel
                coff = (t_abs - gb) * LANES
                nh = cnt[cbase + t_rel]

                def hit_body(h, st):
                    rs, cur, pend = st
                    lp = bins[pl.ds(bbase + t_rel * K + h, 16)][0]
                    lane = lax.bitwise_and(lp, LANES - 1)
                    outp = lax.shift_right_logical(lp, 7)
                    lane = lane + outp * 0
                    rs = rs + 1
                    full = rs == SROWS

                    @pl.when(full)
                    def _():
                        pltpu.async_copy(stage.at[cur],
                                         out_hbm.at[posb.at[cur]], sem_flush)

                        @pl.when(pend == 1)
                        def _():
                            flush_wait()
                        reset_posb_dyn(1 - cur)

                    return (jnp.where(full, 0, rs),
                            jnp.where(full, 1 - cur, cur),
                            jnp.where(full, 1, pend))

                return lax.fori_loop(0, nh, hit_body, st)

            st = lax.fori_loop(0, GRP, tile_body, st)
            fire(g + RING)
            return st
        return lax.fori_loop(0, ngrp, grp_body, st)

    def reset_posb_dyn(c):
        for j in range(SROWS // 16):
            plsc.store_scatter(
                posb.at[c], [lax.iota(jnp.int32, 16) + 16 * j],
                _splat(3 * B + 16 * j) + lax.iota(jnp.int32, 16))

    st = sweep(pt_hbm, t0p, ntp, TP, CNT_P, BIN_P, KP, (0, 0, 0), primed=True)
    rs, cur, pend = sweep(vt_hbm, t0v, ntv, TV, CNT_V, BIN_V, KV, st)

    @pl.when(rs > 0)
    def _():
        pltpu.async_copy(stage.at[cur], out_hbm.at[posb.at[cur]], sem_flush)
    npend = pend + (rs > 0).astype(jnp.int32)

    @pl.when(npend >= 1)
    def _():
        flush_wait()

    @pl.when(npend >= 2)
    def _():
        flush_wait()


@functools.cache
def _sc_sweep_kernel():
    mesh = plsc.VectorSubcoreMesh(core_axis_name="c", subcore_axis_name="s")
    return pl.kernel(
        _sc_sweep_body,
        out_type=jax.ShapeDtypeStruct((OUT_ROWS, LANES), jnp.float32),
        mesh=mesh,
        scratch_types=[
            pltpu.VMEM((2 * SCH,), jnp.int32),
            pltpu.VMEM((RING, D, GRP * LANES), jnp.float32),
            pltpu.VMEM((2, SROWS, LANES), jnp.float32),
            pltpu.VMEM((2, SROWS), jnp.int32),
            pltpu.VMEM((HITCAP + 16,), jnp.int32),
            pltpu.VMEM((HITCAP + 16,), jnp.int32),
            pltpu.VMEM((BINS + 16,), jnp.int32),
            pltpu.SMEM((NCNT,), jnp.int32),
            pltpu.SemaphoreType.DMA,
            pltpu.SemaphoreType.DMA,
            pltpu.SemaphoreType.DMA,
        ],
        compiler_params=pltpu.CompilerParams(
            use_tc_tiling_on_sc=True, needs_layout_passes=False),
    )


BS = 2048
GRID = B // BS
_SQRT_HALF = 0.7071067811865476


def _gelu(x):
    return 0.5 * x * (1.0 + lax.erf(x * _SQRT_HALF))


def _mlp_body(p_ref, vw_ref, vl_ref, w1_ref, b1_ref, w2_ref, b2_ref,
              w3_ref, b3_ref, rw_ref, rl_ref):
    w1a = w1_ref[:D, :]
    w1b = w1_ref[D:, :]
    pa = jnp.dot(p_ref[:, :D], w1a, preferred_element_type=jnp.float32) + b1_ref[...]

    def head(v):
        h = _gelu(pa + jnp.dot(v, w1b, preferred_element_type=jnp.float32))
        h = _gelu(jnp.dot(h, w2_ref[...], preferred_element_type=jnp.float32)
                  + b2_ref[...])
        r = jnp.dot(h, w3_ref[...], preferred_element_type=jnp.float32)
        return r[:, 0] + b3_ref[0, 0]

    rw_ref[...] = head(vw_ref[:, :D])
    rl_ref[...] = head(vl_ref[:, :D])


def _mlp(g, W1, b1, W2, b2, W3, b3):
    full = lambda shape: pl.BlockSpec(shape, lambda i: tuple(0 for _ in shape))
    return pl.pallas_call(
        _mlp_body,
        grid=(GRID,),
        in_specs=[
            pl.BlockSpec((BS, LANES), lambda i: (i, 0)),
            pl.BlockSpec((BS, LANES), lambda i: (i + B // BS, 0)),
            pl.BlockSpec((BS, LANES), lambda i: (i + 2 * (B // BS), 0)),
            full((2 * D, H)), full((1, H)),
            full((H, H)), full((1, H)),
            full((H, 1)), full((1, 1)),
        ],
        out_specs=[pl.BlockSpec((BS,), lambda i: (i,))] * 2,
        out_shape=[jax.ShapeDtypeStruct((B,), jnp.float32)] * 2,
    )(g, g, g, W1, b1, W2, b2, W3, b3)


@jax.jit
def kernel(prompt_idx, preferred_idx, rejected_idx, video_emb, prompt_emb,
           W1, b1, W2, b2, W3, b3):
    g = _sc_sweep_kernel()(prompt_idx, preferred_idx, rejected_idx,
                           prompt_emb.T, video_emb.T)
    r_w, r_l = _mlp(g, W1, b1.reshape(1, H), W2, b2.reshape(1, H),
                    W3, b3.reshape(1, 1))
    return r_w, r_l


# speedup vs baseline: 2.6424x; 2.6424x over previous
import functools

import jax
import jax.numpy as jnp
from jax import lax
from jax.experimental import pallas as pl
from jax.experimental.pallas import tpu as pltpu
from jax.experimental.pallas import tpu_sc as plsc

B = 16384
D = 64
H = 128
NV = 1000000
NP = 100000

NC = 2
NS = 16
NW = NC * NS

LANES = 128
TV = (NV + LANES - 1) // LANES
TP = (NP + LANES - 1) // LANES
TV_Q, TV_R = divmod(TV, NW)
TP_Q, TP_R = divmod(TP, NW)
NT_V = TV_Q + 1
NT_P = TP_Q + 1

KP = 64
KV = 26
BIN_P = 0
BIN_V = NT_P * KP
BINS = BIN_V + NT_V * KV
TRASH = BINS

HITCAP_P = 768
HITCAP_V = 1408
HITCAP = HITCAP_P + HITCAP_V

CNT_P = 0
CNT_V = NT_P + 8
NCNT = CNT_V + NT_V + 8

OUT_ROWS = 3 * B + LANES
RING = 3
GRP = 4
SCH = 2048
SROWS = 56


def _splat(x):
    return jnp.full((16,), x, jnp.int32)


def _sc_sweep_body(pidx_hbm, widx_hbm, lidx_hbm, pt_hbm, vt_hbm, out_hbm,
                   idxv, panels, stage, posb, hit_t, hit_lp, bins, cnt,
                   sem_idx, sem_ring, sem_flush):
    wid = lax.axis_index("s") * NC + lax.axis_index("c")
    t0v = wid * TV_Q + jnp.minimum(wid, TV_R)
    ntv = TV_Q + (wid < TV_R).astype(jnp.int32)
    t0p = wid * TP_Q + jnp.minimum(wid, TP_R)
    ntp = TP_Q + (wid < TP_R).astype(jnp.int32)

    for g in range(RING):
        pltpu.async_copy(
            pt_hbm.at[:, pl.ds(jnp.minimum(t0p + g * GRP, TP - GRP) * LANES,
                               GRP * LANES)],
            panels.at[g], sem_ring)

    def zcnt(i, _):
        cnt[i] = 0
        return ()
    lax.fori_loop(0, NCNT, zcnt, ())

    def scan(src_hbm, t0, nt, pos_base, off0):
        pltpu.async_copy(src_hbm.at[pl.ds(0, SCH)], idxv.at[pl.ds(0, SCH)],
                         sem_idx)

        def chunk(cix, off):
            pltpu.make_async_copy(src_hbm.at[pl.ds(0, SCH)],
                                  idxv.at[pl.ds(0, SCH)], sem_idx).wait()

            @pl.when(cix + 1 < B // SCH)
            def _():
                pltpu.async_copy(
                    src_hbm.at[pl.ds((cix + 1) * SCH, SCH)],
                    idxv.at[pl.ds(lax.rem(cix + 1, 2) * SCH, SCH)], sem_idx)
            cbase = lax.rem(cix, 2) * SCH

            def body(k, off):
                iv0 = idxv[pl.ds(cbase + k * 32, 16)]
                iv1 = idxv[pl.ds(cbase + k * 32 + 16, 16)]
                t_0 = lax.shift_right_logical(iv0, 7)
                t_1 = lax.shift_right_logical(iv1, 7)
                m0 = (t_0 >= t0) & (t_0 < t0 + nt)
                m1 = (t_1 >= t0) & (t_1 < t0 + nt)
                c0 = plsc.cumsum(m0.astype(jnp.int32))
                c1 = plsc.cumsum(m1.astype(jnp.int32))
                base = pos_base + cix * SCH + k * 32 + lax.iota(jnp.int32, 16)
                pos0 = jnp.where(m0, jnp.minimum(off + c0 - 1, HITCAP), HITCAP)
                plsc.store_scatter(hit_t, [pos0], t_0)
                plsc.store_scatter(hit_lp, [pos0],
                                   lax.bitwise_and(iv0, LANES - 1) + (base << 7))
                off1 = off + c0[15]
                pos1 = jnp.where(m1, jnp.minimum(off1 + c1 - 1, HITCAP), HITCAP)
                plsc.store_scatter(hit_t, [pos1], t_1)
                plsc.store_scatter(hit_lp, [pos1],
                                   lax.bitwise_and(iv1, LANES - 1)
                                   + ((base + 16) << 7))
                return off1 + c1[15]
            return lax.fori_loop(0, SCH // 32, body, off)
        return lax.fori_loop(0, B // SCH, chunk, off0)

    nh_p = scan(pidx_hbm, t0p, ntp, 0, 0)
    nh_w = scan(widx_hbm, t0v, ntv, B, HITCAP_P)
    nh_v = scan(lidx_hbm, t0v, ntv, 2 * B, nh_w)

    def binhits(lo, hi, t0, cbase, bbase, K):
        def body(h, _):
            t = hit_t[pl.ds(h, 16)][0]
            lp = hit_lp[pl.ds(h, 16)][0]
            r = t - t0
            slot = cnt[cbase + r]
            cnt[cbase + r] = slot + 1
            addr = jnp.where(slot < K, bbase + r * K + slot, TRASH)
            plsc.store_scatter(bins, [_splat(addr)], _splat(lp))
            return ()
        lax.fori_loop(lo, hi, body, ())

    binhits(0, nh_p, t0p, CNT_P, BIN_P, KP)
    binhits(HITCAP_P, nh_v, t0v, CNT_V, BIN_V, KV)

    def reset_posb(c):
        for j in range(SROWS // 16):
            posb[c, pl.ds(16 * j, 16)] = _splat(3 * B + 16 * j) + lax.iota(jnp.int32, 16)
    reset_posb(0)
    reset_posb(1)

    def flush_wait():
        pltpu.make_async_copy(
            pt_hbm.at[pl.ds(0, SROWS), pl.ds(0, LANES)], stage.at[0],
            sem_flush).wait()

    def sweep(table, t0, nt, tmax, cbase, bbase, K, st, primed=False):
        ngrp = (nt + GRP - 1) // GRP

        def gbase(g):
            return jnp.minimum(t0 + g * GRP, tmax - GRP)

        def fire(g):
            @pl.when(g < ngrp)
            def _():
                pltpu.async_copy(
                    table.at[:, pl.ds(gbase(g) * LANES, GRP * LANES)],
                    panels.at[lax.rem(g, RING)], sem_ring)
        if not primed:
            fire(0)
            fire(1)
            fire(2)

        def grp_body(g, st):
            pltpu.make_async_copy(
                table.at[:, pl.ds(0, GRP * LANES)], panels.at[0],
                sem_ring).wait()
            gmod = lax.rem(g, RING)
            gb = gbase(g)

            def tile_body(tt, st):
                t_rel = g * GRP + tt
                t_abs = t0 + t_rel
                coff = (t_abs - gb) * LANES
                nh = cnt[cbase + t_rel]

                def hit_body(h, st):
                    rs, cur, pend = st
                    lp = bins[pl.ds(bbase + t_rel * K + h, 16)][0]
                    lane = lax.bitwise_and(lp, LANES - 1)
                    outp = lax.shift_right_logical(lp, 7)
                    for j in range(D // 16):
                        fv = lax.iota(jnp.int32, 16) + (16 * j)
                        x = plsc.load_gather(
                            panels, [_splat(gmod), fv, _splat(coff + lane)])
                        plsc.store_scatter(stage,
                                           [_splat(cur), _splat(rs), fv + 0], x)
                    plsc.store_scatter(posb.at[cur], [_splat(rs)], _splat(outp))
                    rs = rs + 1
                    full = rs == SROWS

                    @pl.when(full)
                    def _():
                        pltpu.async_copy(stage.at[cur],
                                         out_hbm.at[posb.at[cur]], sem_flush)

                        @pl.when(pend == 1)
                        def _():
                            flush_wait()
                        reset_posb_dyn(1 - cur)

                    return (jnp.where(full, 0, rs),
                            jnp.where(full, 1 - cur, cur),
                            jnp.where(full, 1, pend))

                return lax.fori_loop(0, nh, hit_body, st)

            st = lax.fori_loop(0, GRP, tile_body, st)
            fire(g + RING)
            return st
        return lax.fori_loop(0, ngrp, grp_body, st)

    def reset_posb_dyn(c):
        for j in range(SROWS // 16):
            plsc.store_scatter(
                posb.at[c], [lax.iota(jnp.int32, 16) + 16 * j],
                _splat(3 * B + 16 * j) + lax.iota(jnp.int32, 16))

    st = sweep(pt_hbm, t0p, ntp, TP, CNT_P, BIN_P, KP, (0, 0, 0), primed=True)
    rs, cur, pend = sweep(vt_hbm, t0v, ntv, TV, CNT_V, BIN_V, KV, st)

    @pl.when(rs > 0)
    def _():
        pltpu.async_copy(stage.at[cur], out_hbm.at[posb.at[cur]], sem_flush)
    npend = pend + (rs > 0).astype(jnp.int32)

    @pl.when(npend >= 1)
    def _():
        flush_wait()

    @pl.when(npend >= 2)
    def _():
        flush_wait()


@functools.cache
def _sc_sweep_kernel():
    mesh = plsc.VectorSubcoreMesh(core_axis_name="c", subcore_axis_name="s")
    return pl.kernel(
        _sc_sweep_body,
        out_type=jax.ShapeDtypeStruct((OUT_ROWS, LANES), jnp.float32),
        mesh=mesh,
        scratch_types=[
            pltpu.VMEM((2 * SCH,), jnp.int32),
            pltpu.VMEM((RING, D, GRP * LANES), jnp.float32),
            pltpu.VMEM((2, SROWS, LANES), jnp.float32),
            pltpu.VMEM((2, SROWS), jnp.int32),
            pltpu.VMEM((HITCAP + 16,), jnp.int32),
            pltpu.VMEM((HITCAP + 16,), jnp.int32),
            pltpu.VMEM((BINS + 16,), jnp.int32),
            pltpu.SMEM((NCNT,), jnp.int32),
            pltpu.SemaphoreType.DMA,
            pltpu.SemaphoreType.DMA,
            pltpu.SemaphoreType.DMA,
        ],
        compiler_params=pltpu.CompilerParams(
            use_tc_tiling_on_sc=True, needs_layout_passes=False),
    )


BS = 2048
GRID = B // BS
_SQRT_HALF = 0.7071067811865476


def _gelu(x):
    return 0.5 * x * (1.0 + lax.erf(x * _SQRT_HALF))


def _mlp_body(p_ref, vw_ref, vl_ref, w1_ref, b1_ref, w2_ref, b2_ref,
              w3_ref, b3_ref, rw_ref, rl_ref):
    w1a = w1_ref[:D, :]
    w1b = w1_ref[D:, :]
    pa = jnp.dot(p_ref[:, :D], w1a, preferred_element_type=jnp.float32) + b1_ref[...]
    v2 = jnp.concatenate([vw_ref[:, :D], vl_ref[:, :D]], axis=0)
    pa2 = jnp.concatenate([pa, pa], axis=0)
    h = _gelu(pa2 + jnp.dot(v2, w1b, preferred_element_type=jnp.float32))
    h = _gelu(jnp.dot(h, w2_ref[...], preferred_element_type=jnp.float32)
              + b2_ref[...])
    r = jnp.dot(h, w3_ref[...], preferred_element_type=jnp.float32)[:, 0] + b3_ref[0, 0]
    rw_ref[...] = r[:BS]
    rl_ref[...] = r[BS:]


def _mlp(g, W1, b1, W2, b2, W3, b3):
    full = lambda shape: pl.BlockSpec(shape, lambda i: tuple(0 for _ in shape))
    return pl.pallas_call(
        _mlp_body,
        grid=(GRID,),
        in_specs=[
            pl.BlockSpec((BS, LANES), lambda i: (i, 0)),
            pl.BlockSpec((BS, LANES), lambda i: (i + B // BS, 0)),
            pl.BlockSpec((BS, LANES), lambda i: (i + 2 * (B // BS), 0)),
            full((2 * D, H)), full((1, H)),
            full((H, H)), full((1, H)),
            full((H, 1)), full((1, 1)),
        ],
        out_specs=[pl.BlockSpec((BS,), lambda i: (i,))] * 2,
        out_shape=[jax.ShapeDtypeStruct((B,), jnp.float32)] * 2,
    )(g, g, g, W1, b1, W2, b2, W3, b3)


@jax.jit
def kernel(prompt_idx, preferred_idx, rejected_idx, video_emb, prompt_emb,
           W1, b1, W2, b2, W3, b3):
    g = _sc_sweep_kernel()(prompt_idx, preferred_idx, rejected_idx,
                           prompt_emb.T, video_emb.T)
    r_w, r_l = _mlp(g, W1, b1.reshape(1, H), W2, b2.reshape(1, H),
                    W3, b3.reshape(1, 1))
    return r_w, r_l


# speedup vs baseline: 2.6521x; 1.0037x over previous
import functools

import jax
import jax.numpy as jnp
from jax import lax
from jax.experimental import pallas as pl
from jax.experimental.pallas import tpu as pltpu
from jax.experimental.pallas import tpu_sc as plsc

B = 16384
D = 64
H = 128
NV = 1000000
NP = 100000

NC = 2
NS = 16
NW = NC * NS

LANES = 128
TV = (NV + LANES - 1) // LANES
TP = (NP + LANES - 1) // LANES
TV_Q, TV_R = divmod(TV, NW)
TP_Q, TP_R = divmod(TP, NW)
NT_V = TV_Q + 1
NT_P = TP_Q + 1

KP = 64
KV = 26
BIN_P = 0
BIN_V = NT_P * KP
BINS = BIN_V + NT_V * KV
TRASH = BINS

HITCAP_P = 768
HITCAP_V = 1408
HITCAP = HITCAP_P + HITCAP_V

CNT_P = 0
CNT_V = NT_P + 8
NCNT = CNT_V + NT_V + 8

OUT_ROWS = 3 * B + LANES
RING = 3
GRP = 4
SCH = 2048
SROWS = 56


def _splat(x):
    return jnp.full((16,), x, jnp.int32)


def _sc_sweep_body(pidx_hbm, widx_hbm, lidx_hbm, pt_hbm, vt_hbm, out_hbm,
                   idxv, panels, stage, posb, hit_t, hit_lp, bins, cnt,
                   sem_idx, sem_ring, sem_flush):
    wid = lax.axis_index("s") * NC + lax.axis_index("c")
    t0v = wid * TV_Q + jnp.minimum(wid, TV_R)
    ntv = TV_Q + (wid < TV_R).astype(jnp.int32)
    t0p = wid * TP_Q + jnp.minimum(wid, TP_R)
    ntp = TP_Q + (wid < TP_R).astype(jnp.int32)

    for g in range(RING):
        pltpu.async_copy(
            pt_hbm.at[:, pl.ds(jnp.minimum(t0p + g * GRP, TP - GRP) * LANES,
                               GRP * LANES)],
            panels.at[g], sem_ring)

    def zcnt(i, _):
        cnt[i] = 0
        return ()
    lax.fori_loop(0, NCNT, zcnt, ())

    def scan(src_hbm, t0, nt, pos_base, off0):
        pltpu.async_copy(src_hbm.at[pl.ds(0, SCH)], idxv.at[pl.ds(0, SCH)],
                         sem_idx)

        def chunk(cix, off):
            pltpu.make_async_copy(src_hbm.at[pl.ds(0, SCH)],
                                  idxv.at[pl.ds(0, SCH)], sem_idx).wait()

            @pl.when(cix + 1 < B // SCH)
            def _():
                pltpu.async_copy(
                    src_hbm.at[pl.ds((cix + 1) * SCH, SCH)],
                    idxv.at[pl.ds(lax.rem(cix + 1, 2) * SCH, SCH)], sem_idx)
            cbase = lax.rem(cix, 2) * SCH

            def body(k, off):
                iv0 = idxv[pl.ds(cbase + k * 32, 16)]
                iv1 = idxv[pl.ds(cbase + k * 32 + 16, 16)]
                t_0 = lax.shift_right_logical(iv0, 7)
                t_1 = lax.shift_right_logical(iv1, 7)
                m0 = (t_0 >= t0) & (t_0 < t0 + nt)
                m1 = (t_1 >= t0) & (t_1 < t0 + nt)
                c0 = plsc.cumsum(m0.astype(jnp.int32))
                c1 = plsc.cumsum(m1.astype(jnp.int32))
                base = pos_base + cix * SCH + k * 32 + lax.iota(jnp.int32, 16)
                pos0 = jnp.where(m0, jnp.minimum(off + c0 - 1, HITCAP), HITCAP)
                plsc.store_scatter(hit_t, [pos0], t_0)
                plsc.store_scatter(hit_lp, [pos0],
                                   lax.bitwise_and(iv0, LANES - 1) + (base << 7))
                off1 = off + c0[15]
                pos1 = jnp.where(m1, jnp.minimum(off1 + c1 - 1, HITCAP), HITCAP)
                plsc.store_scatter(hit_t, [pos1], t_1)
                plsc.store_scatter(hit_lp, [pos1],
                                   lax.bitwise_and(iv1, LANES - 1)
                                   + ((base + 16) << 7))
                return off1 + c1[15]
            return lax.fori_loop(0, SCH // 32, body, off)
        return lax.fori_loop(0, B // SCH, chunk, off0)

    with jax.named_scope("ph_scan"):
        nh_p = scan(pidx_hbm, t0p, ntp, 0, 0)
        nh_w = scan(widx_hbm, t0v, ntv, B, HITCAP_P)
        nh_v = scan(lidx_hbm, t0v, ntv, 2 * B, nh_w)

    def binhits(lo, hi, t0, cbase, bbase, K):
        def body(h, _):
            t = hit_t[pl.ds(h, 16)][0]
            lp = hit_lp[pl.ds(h, 16)][0]
            r = t - t0
            slot = cnt[cbase + r]
            cnt[cbase + r] = slot + 1
            addr = jnp.where(slot < K, bbase + r * K + slot, TRASH)
            plsc.store_scatter(bins, [_splat(addr)], _splat(lp))
            return ()
        lax.fori_loop(lo, hi, body, ())

    with jax.named_scope("ph_bin"):
        binhits(0, nh_p, t0p, CNT_P, BIN_P, KP)
        binhits(HITCAP_P, nh_v, t0v, CNT_V, BIN_V, KV)

    def reset_posb(c):
        for j in range(SROWS // 16):
            posb[c, pl.ds(16 * j, 16)] = _splat(3 * B + 16 * j) + lax.iota(jnp.int32, 16)
    reset_posb(0)
    reset_posb(1)

    def flush_wait():
        pltpu.make_async_copy(
            pt_hbm.at[pl.ds(0, SROWS), pl.ds(0, LANES)], stage.at[0],
            sem_flush).wait()

    def sweep(table, t0, nt, tmax, cbase, bbase, K, st, primed=False):
        ngrp = (nt + GRP - 1) // GRP

        def gbase(g):
            return jnp.minimum(t0 + g * GRP, tmax - GRP)

        def fire(g):
            @pl.when(g < ngrp)
            def _():
                pltpu.async_copy(
                    table.at[:, pl.ds(gbase(g) * LANES, GRP * LANES)],
                    panels.at[lax.rem(g, RING)], sem_ring)
        if not primed:
            fire(0)
            fire(1)
            fire(2)

        def grp_body(g, st):
            pltpu.make_async_copy(
                table.at[:, pl.ds(0, GRP * LANES)], panels.at[0],
                sem_ring).wait()
            gmod = lax.rem(g, RING)
            gb = gbase(g)

            def tile_body(tt, st):
                t_rel = g * GRP + tt
                t_abs = t0 + t_rel
                coff = (t_abs - gb) * LANES
                nh = cnt[cbase + t_rel]

                def hit_body(h, st):
                    rs, cur, pend = st
                    lp = bins[pl.ds(bbase + t_rel * K + h, 16)][0]
                    lane = lax.bitwise_and(lp, LANES - 1)
                    outp = lax.shift_right_logical(lp, 7)
                    for j in range(D // 16):
                        fv = lax.iota(jnp.int32, 16) + (16 * j)
                        x = plsc.load_gather(
                            panels, [_splat(gmod), fv, _splat(coff + lane)])
                        plsc.store_scatter(stage,
                                           [_splat(cur), _splat(rs), fv + 0], x)
                    plsc.store_scatter(posb.at[cur], [_splat(rs)], _splat(outp))
                    rs = rs + 1
                    full = rs == SROWS

                    @pl.when(full)
                    def _():
                        pltpu.async_copy(stage.at[cur],
                                         out_hbm.at[posb.at[cur]], sem_flush)

                        @pl.when(pend == 1)
                        def _():
                            flush_wait()
                        reset_posb_dyn(1 - cur)

                    return (jnp.where(full, 0, rs),
                            jnp.where(full, 1 - cur, cur),
                            jnp.where(full, 1, pend))

                return lax.fori_loop(0, nh, hit_body, st)

            st = lax.fori_loop(0, GRP, tile_body, st)
            fire(g + RING)
            return st
        return lax.fori_loop(0, ngrp, grp_body, st)

    def reset_posb_dyn(c):
        for j in range(SROWS // 16):
            plsc.store_scatter(
                posb.at[c], [lax.iota(jnp.int32, 16) + 16 * j],
                _splat(3 * B + 16 * j) + lax.iota(jnp.int32, 16))

    with jax.named_scope("ph_sweep_p"):
        st = sweep(pt_hbm, t0p, ntp, TP, CNT_P, BIN_P, KP, (0, 0, 0), primed=True)
    with jax.named_scope("ph_sweep_v"):
        rs, cur, pend = sweep(vt_hbm, t0v, ntv, TV, CNT_V, BIN_V, KV, st)

    @pl.when(rs > 0)
    def _():
        pltpu.async_copy(stage.at[cur], out_hbm.at[posb.at[cur]], sem_flush)
    npend = pend + (rs > 0).astype(jnp.int32)

    @pl.when(npend >= 1)
    def _():
        flush_wait()

    @pl.when(npend >= 2)
    def _():
        flush_wait()


@functools.cache
def _sc_sweep_kernel():
    mesh = plsc.VectorSubcoreMesh(core_axis_name="c", subcore_axis_name="s")
    return pl.kernel(
        _sc_sweep_body,
        out_type=jax.ShapeDtypeStruct((OUT_ROWS, LANES), jnp.float32),
        mesh=mesh,
        scratch_types=[
            pltpu.VMEM((2 * SCH,), jnp.int32),
            pltpu.VMEM((RING, D, GRP * LANES), jnp.float32),
            pltpu.VMEM((2, SROWS, LANES), jnp.float32),
            pltpu.VMEM((2, SROWS), jnp.int32),
            pltpu.VMEM((HITCAP + 16,), jnp.int32),
            pltpu.VMEM((HITCAP + 16,), jnp.int32),
            pltpu.VMEM((BINS + 16,), jnp.int32),
            pltpu.SMEM((NCNT,), jnp.int32),
            pltpu.SemaphoreType.DMA,
            pltpu.SemaphoreType.DMA,
            pltpu.SemaphoreType.DMA,
        ],
        compiler_params=pltpu.CompilerParams(
            use_tc_tiling_on_sc=True, needs_layout_passes=False),
    )


BS = 2048
GRID = B // BS
_SQRT_HALF = 0.7071067811865476


def _gelu(x):
    return 0.5 * x * (1.0 + lax.erf(x * _SQRT_HALF))


def _mlp_body(p_ref, vw_ref, vl_ref, w1_ref, b1_ref, w2_ref, b2_ref,
              w3_ref, b3_ref, rw_ref, rl_ref):
    w1a = w1_ref[:D, :]
    w1b = w1_ref[D:, :]
    pa = jnp.dot(p_ref[:, :D], w1a, preferred_element_type=jnp.float32) + b1_ref[...]

    def head(v):
        h = _gelu(pa + jnp.dot(v, w1b, preferred_element_type=jnp.float32))
        h = _gelu(jnp.dot(h, w2_ref[...], preferred_element_type=jnp.float32)
                  + b2_ref[...])
        r = jnp.dot(h, w3_ref[...], preferred_element_type=jnp.float32)
        return r[:, 0] + b3_ref[0, 0]

    rw_ref[...] = head(vw_ref[:, :D])
    rl_ref[...] = head(vl_ref[:, :D])


def _mlp(g, W1, b1, W2, b2, W3, b3):
    full = lambda shape: pl.BlockSpec(shape, lambda i: tuple(0 for _ in shape))
    return pl.pallas_call(
        _mlp_body,
        grid=(GRID,),
        in_specs=[
            pl.BlockSpec((BS, LANES), lambda i: (i, 0)),
            pl.BlockSpec((BS, LANES), lambda i: (i + B // BS, 0)),
            pl.BlockSpec((BS, LANES), lambda i: (i + 2 * (B // BS), 0)),
            full((2 * D, H)), full((1, H)),
            full((H, H)), full((1, H)),
            full((H, 1)), full((1, 1)),
        ],
        out_specs=[pl.BlockSpec((BS,), lambda i: (i,))] * 2,
        out_shape=[jax.ShapeDtypeStruct((B,), jnp.float32)] * 2,
    )(g, g, g, W1, b1, W2, b2, W3, b3)


@jax.jit
def kernel(prompt_idx, preferred_idx, rejected_idx, video_emb, prompt_emb,
           W1, b1, W2, b2, W3, b3):
    g = _sc_sweep_kernel()(prompt_idx, preferred_idx, rejected_idx,
                           prompt_emb.T, video_emb.T)
    r_w, r_l = _mlp(g, W1, b1.reshape(1, H), W2, b2.reshape(1, H),
                    W3, b3.reshape(1, 1))
    return r_w, r_l
